# baseline (speedup 1.0000x reference)
import functools

import jax
import jax.numpy as jnp
from jax import lax
from jax.experimental import pallas as pl
from jax.experimental.pallas import tpu as pltpu
from jax.experimental.pallas import tpu_sc as plsc

NV = 10000
D = 16
E = 320000
NC = 2
NS = 16
NW = NC * NS
EPW = E // NW
GC = 125
NG = EPW // GC
GRP = 8
GW = GC * GRP
NGRP = NG // GRP
EC = 125
NCHUNK = E // EC
CPW = NCHUNK // NW
BLK = 512
ZT = 10
ZR = NV // ZT

_SC_PARAMS = pltpu.CompilerParams(
    needs_layout_passes=False, use_tc_tiling_on_sc=False)


def _sc_gather(x2d, srcs):
    mesh = plsc.VectorSubcoreMesh(core_axis_name="c", subcore_axis_name="s")

    @functools.partial(
        pl.kernel,
        mesh=mesh,
        compiler_params=_SC_PARAMS,
        out_type=jax.ShapeDtypeStruct((D, E), jnp.float32),
        scratch_types=[
            pltpu.VMEM((NG, GC), jnp.int32),
            pltpu.VMEM((GW, D), jnp.float32),
            pltpu.VMEM((GW, D), jnp.float32),
            pltpu.VMEM((D * GW,), jnp.float32),
            pltpu.SemaphoreType.DMA,
            pltpu.SemaphoreType.DMA,
            pltpu.SemaphoreType.DMA,
        ],
    )
    def gather_fn(x_hbm, srcs_hbm, out_hbm, srcs_v, rows_a, rows_b,
                  rowst_v, sem_a, sem_b, sem_out):
        cid = lax.axis_index("c")
        sid = lax.axis_index("s")
        wid = sid * NC + cid
        base = wid * EPW
        pltpu.sync_copy(srcs_hbm.at[wid], srcs_v)
        iota = lax.iota(jnp.int32, D)
        iota_gw = iota * GW
        rows = (rows_a, rows_b)
        sems = (sem_a, sem_b)

        def fire(g, buf, sem):
            return [
                pltpu.async_copy(
                    x_hbm.at[srcs_v.at[g * GRP + k]],
                    buf.at[pl.ds(k * GC, GC)], sem)
                for k in range(GRP)
            ]

        pend_out = []
        pend = {0: fire(0, rows[0], sems[0])}
        for g in range(NGRP):
            if g + 1 < NGRP:
                pend[g + 1] = fire(g + 1, rows[(g + 1) % 2], sems[(g + 1) % 2])
            for cp in pend.pop(g):
                cp.wait()
            for cp in pend_out:
                cp.wait()
            pend_out = []
            buf = rows[g % 2]

            def edge_body(e, ecarry, _buf=buf):
                for u in range(4):
                    vec = _buf[e * 4 + u, :]
                    plsc.store_scatter(rowst_v, [iota_gw + (e * 4 + u)], vec)
                return ecarry

            lax.fori_loop(0, GW // 4, edge_body, 0)
            pend_out = [
                pltpu.async_copy(
                    rowst_v.at[pl.ds(j * GW, GW)],
                    out_hbm.at[j, pl.ds(base + g * GW, GW)], sem_out)
                for j in range(D)
            ]
        for cp in pend_out:
            cp.wait()

    return gather_fn(x2d, srcs)


def _tc_messages(wt, xst, bt):
    def tc_body(wt_ref, xst_ref, bt_ref, o_ref):
        xt = xst_ref[...]
        msg_t = bt_ref[...]
        for j in range(D):
            msg_t = msg_t + wt_ref[:, j, :] * xt[j, :][None, :]
        o_ref[...] = msg_t

    return pl.pallas_call(
        tc_body,
        grid=(E // BLK,),
        in_specs=[
            pl.BlockSpec((D, D, BLK), lambda i: (0, 0, i)),
            pl.BlockSpec((D, BLK), lambda i: (0, i)),
            pl.BlockSpec((D, BLK), lambda i: (0, i)),
        ],
        out_specs=pl.BlockSpec((D, BLK), lambda i: (0, i)),
        out_shape=jax.ShapeDtypeStruct((D, E), jnp.float32),
    )(wt, xst, bt)


def _sc_scatter(msg3, dsts2, zeros):
    mesh = plsc.VectorSubcoreMesh(core_axis_name="c", subcore_axis_name="s")

    @functools.partial(
        pl.kernel,
        mesh=mesh,
        compiler_params=_SC_PARAMS,
        out_type=jax.ShapeDtypeStruct((NC, NV, D), jnp.float32),
        scratch_types=[
            pltpu.VMEM((GRP, EC), jnp.int32),
            pltpu.VMEM((GRP, EC), jnp.int32),
            pltpu.VMEM((GRP, D, EC), jnp.float32),
            pltpu.VMEM((GRP, D, EC), jnp.float32),
            pltpu.VMEM((GRP, EC, D), jnp.float32),
            pltpu.VMEM((GRP, EC, D), jnp.float32),
            pltpu.VMEM_SHARED((NV, D), jnp.float32),
            pltpu.SemaphoreType.DMA,
            pltpu.SemaphoreType.DMA,
            pltpu.SemaphoreType.DMA,
            pltpu.SemaphoreType.DMA,
        ],
    )
    def scatter_fn(msg_hbm, dsts_hbm, z_hbm, out_hbm,
                   dst_a, dst_b, mt_a, mt_b, msg_a, msg_b, preds_sh,
                   sem_da, sem_db, sem_sa, sem_sb):
        cid = lax.axis_index("c")
        sid = lax.axis_index("s")
        wid = sid * NC + cid
        @pl.when(sid < ZT)
        def _():
            pltpu.sync_copy(z_hbm.at[pl.ds(sid * ZR, ZR)],
                            preds_sh.at[pl.ds(sid * ZR, ZR)])
        plsc.subcore_barrier()

        ngrp_c = CPW // GRP
        dsts_b_ = (dst_a, dst_b)
        mts_b_ = (mt_a, mt_b)
        msgs_b_ = (msg_a, msg_b)
        sem_d = (sem_da, sem_db)
        sem_s = (sem_sa, sem_sb)
        iota = lax.iota(jnp.int32, D)

        def fire_loads(g, dbuf, mbuf, sem):
            cps = []
            for k in range(GRP):
                chunk = wid + (g * GRP + k) * NW
                cps.append(pltpu.async_copy(
                    dsts_hbm.at[chunk], dbuf.at[k], sem))
                cps.append(pltpu.async_copy(
                    msg_hbm.at[:, chunk], mbuf.at[k], sem))
            return cps

        pend_s = {0: [], 1: []}
        pend = {0: fire_loads(0, dsts_b_[0], mts_b_[0], sem_d[0])}
        for g in range(ngrp_c):
            par = g % 2
            if g + 1 < ngrp_c:
                npar = (g + 1) % 2
                for cp in pend_s[npar]:
                    cp.wait()
                pend_s[npar] = []
                pend[g + 1] = fire_loads(
                    g + 1, dsts_b_[npar], mts_b_[npar], sem_d[npar])
            for cp in pend.pop(g):
                cp.wait()
            for k in range(GRP):
                mt_k = mts_b_[par].at[k]
                msg_k = msgs_b_[par].at[k]

                def edge_body(e, ecarry, _mt=mt_k, _msg=msg_k):
                    for u in range(5):
                        ee = e * 5 + u
                        vec = plsc.load_gather(
                            _mt, [iota, jnp.full((D,), ee, jnp.int32)])
                        _msg[ee, :] = vec
                    return ecarry

                lax.fori_loop(0, EC // 5, edge_body, 0)
            for k in range(GRP):
                pend_s[par].append(pltpu.async_copy(
                    msgs_b_[par].at[k], preds_sh.at[dsts_b_[par].at[k]],
                    sem_s[par], add=True))
        for par in (0, 1):
            for cp in pend_s[par]:
                cp.wait()
        plsc.subcore_barrier()
        @pl.when(sid < ZT)
        def _():
            pltpu.sync_copy(preds_sh.at[pl.ds(sid * ZR, ZR)],
                            out_hbm.at[cid, pl.ds(sid * ZR, ZR)])

    return scatter_fn(msg3, dsts2, zeros)


def _tc_energies(partials, xw):
    def tc_body(p_ref, x_ref, o_ref):
        r = p_ref[0] + p_ref[1] - x_ref[...]
        r2 = r * r
        li = lax.broadcasted_iota(jnp.int32, (128, 8), 0)
        ci = lax.broadcasted_iota(jnp.int32, (128, 8), 1)
        g = (li // D == ci).astype(jnp.float32)
        o_ref[...] = jnp.dot(r2, g, preferred_element_type=jnp.float32)

    return pl.pallas_call(
        tc_body,
        out_shape=jax.ShapeDtypeStruct((1250, 8), jnp.float32),
    )(partials, xw)


def kernel(x, edge_index, W, b):
    x2d = x.reshape(NV, D)
    srcs = edge_index[0].astype(jnp.int32).reshape(NW, NG, GC)
    dsts2 = edge_index[1].astype(jnp.int32).reshape(NCHUNK, EC)
    wt = W.transpose(1, 2, 0)
    xst = _sc_gather(x2d, srcs)
    msg_t = _tc_messages(wt, xst, b.transpose(1, 0))
    zeros = jnp.zeros((NV, D), jnp.float32)
    partials = _sc_scatter(msg_t.reshape(D, NCHUNK, EC), dsts2, zeros)
    energies = _tc_energies(partials.reshape(NC, 1250, 128),
                            x.reshape(1250, 128))
    return energies.reshape(NV)


# speedup vs baseline: 1.4544x; 1.4544x over previous
import functools

import jax
import jax.numpy as jnp
from jax import lax
from jax.experimental import pallas as pl
from jax.experimental.pallas import tpu as pltpu
from jax.experimental.pallas import tpu_sc as plsc

NV = 10000
D = 16
E = 320000
NC = 2
NS = 16
NW = NC * NS
EPW = E // NW
GC = 125
NG = EPW // GC
GRP = 8
GW = GC * GRP
NGRP = NG // GRP
EC = 125
NCHUNK = E // EC
CPW = NCHUNK // NW
BLK = 2560
SUB = 512
ZT = 10
ZR = NV // ZT

_SC_PARAMS = pltpu.CompilerParams(
    needs_layout_passes=False, use_tc_tiling_on_sc=False)


def _sc_gather(x2d, srcs):
    mesh = plsc.VectorSubcoreMesh(core_axis_name="c", subcore_axis_name="s")

    @functools.partial(
        pl.kernel,
        mesh=mesh,
        compiler_params=_SC_PARAMS,
        out_type=jax.ShapeDtypeStruct((D, E), jnp.float32),
        scratch_types=[
            pltpu.VMEM((NG, GC), jnp.int32),
            pltpu.VMEM((GW, D), jnp.float32),
            pltpu.VMEM((GW, D), jnp.float32),
            pltpu.VMEM((D * GW,), jnp.float32),
            pltpu.SemaphoreType.DMA,
            pltpu.SemaphoreType.DMA,
            pltpu.SemaphoreType.DMA,
        ],
    )
    def gather_fn(x_hbm, srcs_hbm, out_hbm, srcs_v, rows_a, rows_b,
                  rowst_v, sem_a, sem_b, sem_out):
        cid = lax.axis_index("c")
        sid = lax.axis_index("s")
        wid = sid * NC + cid
        base = wid * EPW
        pltpu.sync_copy(srcs_hbm.at[wid], srcs_v)
        iota = lax.iota(jnp.int32, D)
        iota_gw = iota * GW
        rows = (rows_a, rows_b)
        sems = (sem_a, sem_b)

        def fire(g, buf, sem):
            return [
                pltpu.async_copy(
                    x_hbm.at[srcs_v.at[g * GRP + k]],
                    buf.at[pl.ds(k * GC, GC)], sem)
                for k in range(GRP)
            ]

        pend_out = []
        pend = {0: fire(0, rows[0], sems[0])}
        for g in range(NGRP):
            if g + 1 < NGRP:
                pend[g + 1] = fire(g + 1, rows[(g + 1) % 2], sems[(g + 1) % 2])
            for cp in pend.pop(g):
                cp.wait()
            for cp in pend_out:
                cp.wait()
            pend_out = []
            buf = rows[g % 2]

            def edge_body(e, ecarry, _buf=buf):
                for u in range(4):
                    vec = _buf[e * 4 + u, :]
                    plsc.store_scatter(rowst_v, [iota_gw + (e * 4 + u)], vec)
                return ecarry

            lax.fori_loop(0, GW // 4, edge_body, 0)
            pend_out = [
                pltpu.async_copy(
                    rowst_v.at[pl.ds(j * GW, GW)],
                    out_hbm.at[j, pl.ds(base + g * GW, GW)], sem_out)
                for j in range(D)
            ]
        for cp in pend_out:
            cp.wait()

    return gather_fn(x2d, srcs)


def _tc_messages(wt, xst, bt):
    def tc_body(wt_ref, xst_ref, bt_ref, o_ref):
        for s in range(BLK // SUB):
            sl = pl.ds(s * SUB, SUB)
            xt = xst_ref[:, sl]
            msg_t = bt_ref[:, sl]
            for j in range(D):
                msg_t = msg_t + wt_ref[:, j, sl] * xt[j, :][None, :]
            o_ref[:, sl] = msg_t

    return pl.pallas_call(
        tc_body,
        grid=(E // BLK,),
        in_specs=[
            pl.BlockSpec((D, D, BLK), lambda i: (0, 0, i)),
            pl.BlockSpec((D, BLK), lambda i: (0, i)),
            pl.BlockSpec((D, BLK), lambda i: (0, i)),
        ],
        out_specs=pl.BlockSpec((D, BLK), lambda i: (0, i)),
        out_shape=jax.ShapeDtypeStruct((D, E), jnp.float32),
    )(wt, xst, bt)


def _sc_scatter(msg3, dsts2, zeros):
    mesh = plsc.VectorSubcoreMesh(core_axis_name="c", subcore_axis_name="s")

    @functools.partial(
        pl.kernel,
        mesh=mesh,
        compiler_params=_SC_PARAMS,
        out_type=jax.ShapeDtypeStruct((NC, NV, D), jnp.float32),
        scratch_types=[
            pltpu.VMEM((GRP, EC), jnp.int32),
            pltpu.VMEM((GRP, EC), jnp.int32),
            pltpu.VMEM((GRP, D, EC), jnp.float32),
            pltpu.VMEM((GRP, D, EC), jnp.float32),
            pltpu.VMEM((GRP, EC, D), jnp.float32),
            pltpu.VMEM((GRP, EC, D), jnp.float32),
            pltpu.VMEM_SHARED((NV, D), jnp.float32),
            pltpu.SemaphoreType.DMA,
            pltpu.SemaphoreType.DMA,
            pltpu.SemaphoreType.DMA,
            pltpu.SemaphoreType.DMA,
        ],
    )
    def scatter_fn(msg_hbm, dsts_hbm, z_hbm, out_hbm,
                   dst_a, dst_b, mt_a, mt_b, msg_a, msg_b, preds_sh,
                   sem_da, sem_db, sem_sa, sem_sb):
        cid = lax.axis_index("c")
        sid = lax.axis_index("s")
        wid = sid * NC + cid
        @pl.when(sid < ZT)
        def _():
            pltpu.sync_copy(z_hbm.at[pl.ds(sid * ZR, ZR)],
                            preds_sh.at[pl.ds(sid * ZR, ZR)])
        plsc.subcore_barrier()

        ngrp_c = CPW // GRP
        dsts_b_ = (dst_a, dst_b)
        mts_b_ = (mt_a, mt_b)
        msgs_b_ = (msg_a, msg_b)
        sem_d = (sem_da, sem_db)
        sem_s = (sem_sa, sem_sb)
        iota = lax.iota(jnp.int32, D)

        def fire_loads(g, dbuf, mbuf, sem):
            cps = []
            for k in range(GRP):
                chunk = wid + (g * GRP + k) * NW
                cps.append(pltpu.async_copy(
                    dsts_hbm.at[chunk], dbuf.at[k], sem))
                cps.append(pltpu.async_copy(
                    msg_hbm.at[:, chunk], mbuf.at[k], sem))
            return cps

        pend_s = {0: [], 1: []}
        pend = {0: fire_loads(0, dsts_b_[0], mts_b_[0], sem_d[0])}
        for g in range(ngrp_c):
            par = g % 2
            if g + 1 < ngrp_c:
                npar = (g + 1) % 2
                for cp in pend_s[npar]:
                    cp.wait()
                pend_s[npar] = []
                pend[g + 1] = fire_loads(
                    g + 1, dsts_b_[npar], mts_b_[npar], sem_d[npar])
            for cp in pend.pop(g):
                cp.wait()
            for k in range(GRP):
                mt_k = mts_b_[par].at[k]
                msg_k = msgs_b_[par].at[k]

                def edge_body(e, ecarry, _mt=mt_k, _msg=msg_k):
                    for u in range(5):
                        ee = e * 5 + u
                        vec = plsc.load_gather(
                            _mt, [iota, jnp.full((D,), ee, jnp.int32)])
                        _msg[ee, :] = vec
                    return ecarry

                lax.fori_loop(0, EC // 5, edge_body, 0)
            for k in range(GRP):
                pend_s[par].append(pltpu.async_copy(
                    msgs_b_[par].at[k], preds_sh.at[dsts_b_[par].at[k]],
                    sem_s[par], add=True))
        for par in (0, 1):
            for cp in pend_s[par]:
                cp.wait()
        plsc.subcore_barrier()
        @pl.when(sid < ZT)
        def _():
            pltpu.sync_copy(preds_sh.at[pl.ds(sid * ZR, ZR)],
                            out_hbm.at[cid, pl.ds(sid * ZR, ZR)])

    return scatter_fn(msg3, dsts2, zeros)


def _tc_energies(partials, xw):
    def tc_body(p_ref, x_ref, o_ref):
        r = p_ref[0] + p_ref[1] - x_ref[...]
        r2 = r * r
        li = lax.broadcasted_iota(jnp.int32, (128, 8), 0)
        ci = lax.broadcasted_iota(jnp.int32, (128, 8), 1)
        g = (li // D == ci).astype(jnp.float32)
        o_ref[...] = jnp.dot(r2, g, preferred_element_type=jnp.float32)

    return pl.pallas_call(
        tc_body,
        out_shape=jax.ShapeDtypeStruct((1250, 8), jnp.float32),
    )(partials, xw)


def kernel(x, edge_index, W, b):
    x2d = x.reshape(NV, D)
    srcs = edge_index[0].astype(jnp.int32).reshape(NW, NG, GC)
    dsts2 = edge_index[1].astype(jnp.int32).reshape(NCHUNK, EC)
    wt = W.transpose(1, 2, 0)
    xst = _sc_gather(x2d, srcs)
    msg_t = _tc_messages(wt, xst, b.transpose(1, 0))
    zeros = jnp.zeros((NV, D), jnp.float32)
    partials = _sc_scatter(msg_t.reshape(D, NCHUNK, EC), dsts2, zeros)
    energies = _tc_energies(partials.reshape(NC, 1250, 128),
                            x.reshape(1250, 128))
    return energies.reshape(NV)


# speedup vs baseline: 1.4662x; 1.0081x over previous
import functools

import jax
import jax.numpy as jnp
from jax import lax
from jax.experimental import pallas as pl
from jax.experimental.pallas import tpu as pltpu
from jax.experimental.pallas import tpu_sc as plsc

NV = 10000
D = 16
E = 320000
NC = 2
NS = 16
NW = NC * NS
EPW = E // NW
GC = 125
NG = EPW // GC
GRP = 8
GW = GC * GRP
NGRP = NG // GRP
EC = 125
NCHUNK = E // EC
CPW = NCHUNK // NW
BLK = 6400
SUB = 640
ZT = 10
ZR = NV // ZT

_SC_PARAMS = pltpu.CompilerParams(
    needs_layout_passes=False, use_tc_tiling_on_sc=False)


def _sc_gather(x2d, srcs):
    mesh = plsc.VectorSubcoreMesh(core_axis_name="c", subcore_axis_name="s")

    @functools.partial(
        pl.kernel,
        mesh=mesh,
        compiler_params=_SC_PARAMS,
        out_type=jax.ShapeDtypeStruct((D, E), jnp.float32),
        scratch_types=[
            pltpu.VMEM((NG, GC), jnp.int32),
            pltpu.VMEM((GW, D), jnp.float32),
            pltpu.VMEM((GW, D), jnp.float32),
            pltpu.VMEM((D * GW,), jnp.float32),
            pltpu.SemaphoreType.DMA,
            pltpu.SemaphoreType.DMA,
            pltpu.SemaphoreType.DMA,
        ],
    )
    def gather_fn(x_hbm, srcs_hbm, out_hbm, srcs_v, rows_a, rows_b,
                  rowst_v, sem_a, sem_b, sem_out):
        cid = lax.axis_index("c")
        sid = lax.axis_index("s")
        wid = sid * NC + cid
        base = wid * EPW
        pltpu.sync_copy(srcs_hbm.at[wid], srcs_v)
        iota = lax.iota(jnp.int32, D)
        iota_gw = iota * GW
        rows = (rows_a, rows_b)
        sems = (sem_a, sem_b)

        def fire(g, buf, sem):
            return [
                pltpu.async_copy(
                    x_hbm.at[srcs_v.at[g * GRP + k]],
                    buf.at[pl.ds(k * GC, GC)], sem)
                for k in range(GRP)
            ]

        pend_out = []
        pend = {0: fire(0, rows[0], sems[0])}
        for g in range(NGRP):
            if g + 1 < NGRP:
                pend[g + 1] = fire(g + 1, rows[(g + 1) % 2], sems[(g + 1) % 2])
            for cp in pend.pop(g):
                cp.wait()
            for cp in pend_out:
                cp.wait()
            pend_out = []
            buf = rows[g % 2]

            def edge_body(e, ecarry, _buf=buf):
                for u in range(4):
                    vec = _buf[e * 4 + u, :]
                    plsc.store_scatter(rowst_v, [iota_gw + (e * 4 + u)], vec)
                return ecarry

            lax.fori_loop(0, GW // 4, edge_body, 0)
            pend_out = [
                pltpu.async_copy(
                    rowst_v.at[pl.ds(j * GW, GW)],
                    out_hbm.at[j, pl.ds(base + g * GW, GW)], sem_out)
                for j in range(D)
            ]
        for cp in pend_out:
            cp.wait()

    return gather_fn(x2d, srcs)


def _tc_messages(wt, xst, bt):
    def tc_body(wt_ref, xst_ref, bt_ref, o_ref):
        for s in range(BLK // SUB):
            sl = pl.ds(s * SUB, SUB)
            xt = xst_ref[:, sl]
            msg_t = bt_ref[:, sl]
            for j in range(D):
                msg_t = msg_t + wt_ref[:, j, sl] * xt[j, :][None, :]
            o_ref[:, sl] = msg_t

    return pl.pallas_call(
        tc_body,
        grid=(E // BLK,),
        in_specs=[
            pl.BlockSpec((D, D, BLK), lambda i: (0, 0, i)),
            pl.BlockSpec((D, BLK), lambda i: (0, i)),
            pl.BlockSpec((D, BLK), lambda i: (0, i)),
        ],
        out_specs=pl.BlockSpec((D, BLK), lambda i: (0, i)),
        out_shape=jax.ShapeDtypeStruct((D, E), jnp.float32),
    )(wt, xst, bt)


def _sc_scatter(msg3, dsts2, zeros):
    mesh = plsc.VectorSubcoreMesh(core_axis_name="c", subcore_axis_name="s")

    @functools.partial(
        pl.kernel,
        mesh=mesh,
        compiler_params=_SC_PARAMS,
        out_type=jax.ShapeDtypeStruct((NC, NV, D), jnp.float32),
        scratch_types=[
            pltpu.VMEM((GRP, EC), jnp.int32),
            pltpu.VMEM((GRP, EC), jnp.int32),
            pltpu.VMEM((GRP, D, EC), jnp.float32),
            pltpu.VMEM((GRP, D, EC), jnp.float32),
            pltpu.VMEM((GRP, EC, D), jnp.float32),
            pltpu.VMEM((GRP, EC, D), jnp.float32),
            pltpu.VMEM_SHARED((NV, D), jnp.float32),
            pltpu.SemaphoreType.DMA,
            pltpu.SemaphoreType.DMA,
            pltpu.SemaphoreType.DMA,
            pltpu.SemaphoreType.DMA,
        ],
    )
    def scatter_fn(msg_hbm, dsts_hbm, z_hbm, out_hbm,
                   dst_a, dst_b, mt_a, mt_b, msg_a, msg_b, preds_sh,
                   sem_da, sem_db, sem_sa, sem_sb):
        cid = lax.axis_index("c")
        sid = lax.axis_index("s")
        wid = sid * NC + cid
        @pl.when(sid < ZT)
        def _():
            pltpu.sync_copy(z_hbm.at[pl.ds(sid * ZR, ZR)],
                            preds_sh.at[pl.ds(sid * ZR, ZR)])
        plsc.subcore_barrier()

        ngrp_c = CPW // GRP
        dsts_b_ = (dst_a, dst_b)
        mts_b_ = (mt_a, mt_b)
        msgs_b_ = (msg_a, msg_b)
        sem_d = (sem_da, sem_db)
        sem_s = (sem_sa, sem_sb)
        iota = lax.iota(jnp.int32, D)

        def fire_loads(g, dbuf, mbuf, sem):
            cps = []
            for k in range(GRP):
                chunk = wid + (g * GRP + k) * NW
                cps.append(pltpu.async_copy(
                    dsts_hbm.at[chunk], dbuf.at[k], sem))
                cps.append(pltpu.async_copy(
                    msg_hbm.at[:, chunk], mbuf.at[k], sem))
            return cps

        pend_s = {0: [], 1: []}
        pend = {0: fire_loads(0, dsts_b_[0], mts_b_[0], sem_d[0])}
        for g in range(ngrp_c):
            par = g % 2
            if g + 1 < ngrp_c:
                npar = (g + 1) % 2
                for cp in pend_s[npar]:
                    cp.wait()
                pend_s[npar] = []
                pend[g + 1] = fire_loads(
                    g + 1, dsts_b_[npar], mts_b_[npar], sem_d[npar])
            for cp in pend.pop(g):
                cp.wait()
            for k in range(GRP):
                mt_k = mts_b_[par].at[k]
                msg_k = msgs_b_[par].at[k]

                def edge_body(e, ecarry, _mt=mt_k, _msg=msg_k):
                    for u in range(5):
                        ee = e * 5 + u
                        vec = plsc.load_gather(
                            _mt, [iota, jnp.full((D,), ee, jnp.int32)])
                        _msg[ee, :] = vec
                    return ecarry

                lax.fori_loop(0, EC // 5, edge_body, 0)
            for k in range(GRP):
                pend_s[par].append(pltpu.async_copy(
                    msgs_b_[par].at[k], preds_sh.at[dsts_b_[par].at[k]],
                    sem_s[par], add=True))
        for par in (0, 1):
            for cp in pend_s[par]:
                cp.wait()
        plsc.subcore_barrier()
        @pl.when(sid < ZT)
        def _():
            pltpu.sync_copy(preds_sh.at[pl.ds(sid * ZR, ZR)],
                            out_hbm.at[cid, pl.ds(sid * ZR, ZR)])

    return scatter_fn(msg3, dsts2, zeros)


def _tc_energies(partials, xw):
    def tc_body(p_ref, x_ref, o_ref):
        r = p_ref[0] + p_ref[1] - x_ref[...]
        r2 = r * r
        li = lax.broadcasted_iota(jnp.int32, (128, 8), 0)
        ci = lax.broadcasted_iota(jnp.int32, (128, 8), 1)
        g = (li // D == ci).astype(jnp.float32)
        o_ref[...] = jnp.dot(r2, g, preferred_element_type=jnp.float32)

    return pl.pallas_call(
        tc_body,
        out_shape=jax.ShapeDtypeStruct((1250, 8), jnp.float32),
    )(partials, xw)


def kernel(x, edge_index, W, b):
    x2d = x.reshape(NV, D)
    srcs = edge_index[0].astype(jnp.int32).reshape(NW, NG, GC)
    dsts2 = edge_index[1].astype(jnp.int32).reshape(NCHUNK, EC)
    wt = W.transpose(1, 2, 0)
    xst = _sc_gather(x2d, srcs)
    msg_t = _tc_messages(wt, xst, b.transpose(1, 0))
    zeros = jnp.zeros((NV, D), jnp.float32)
    partials = _sc_scatter(msg_t.reshape(D, NCHUNK, EC), dsts2, zeros)
    energies = _tc_energies(partials.reshape(NC, 1250, 128),
                            x.reshape(1250, 128))
    return energies.reshape(NV)
